# baseline (speedup 1.0000x reference)
import functools

import jax
import jax.numpy as jnp
import numpy as np
from jax import lax
from jax.experimental import pallas as pl
from jax.experimental.pallas import tpu as pltpu
from jax.experimental.pallas import tpu_sc as plsc

WINDOW = 2048


def _window_start(max_len: int) -> int:
    if max_len == WINDOW:
        return 0
    return int(np.random.RandomState(0).randint(0, max_len - WINDOW + 1))


def _sc_body(start, ddx_hbm, mdx_hbm, p_hbm, oddx_hbm, omdx_hbm, op_hbm, sem):
    cid = lax.axis_index("c")
    sid = lax.axis_index("s")
    wid = sid * 2 + cid
    b = wid // 2

    @pl.when(wid % 2 == 0)
    def _copy_ddx():
        pltpu.async_copy(
            ddx_hbm.at[b, pl.ds(start, WINDOW)], oddx_hbm.at[b], sem
        ).wait()

    @pl.when(wid % 2 == 1)
    def _copy_mdx():
        pltpu.async_copy(
            mdx_hbm.at[b, pl.ds(start, WINDOW)], omdx_hbm.at[b], sem
        ).wait()

    @pl.when(wid == 0)
    def _copy_p():
        pltpu.async_copy(p_hbm.at[:, pl.ds(start, WINDOW)], op_hbm, sem).wait()


@jax.jit
def kernel(ddx, mdx, p):
    batch, max_len, dm = ddx.shape
    start = _window_start(max_len)
    mesh = plsc.VectorSubcoreMesh(core_axis_name="c", subcore_axis_name="s")
    out_type = (
        jax.ShapeDtypeStruct((batch, WINDOW, dm), ddx.dtype),
        jax.ShapeDtypeStruct((batch, WINDOW, dm), mdx.dtype),
        jax.ShapeDtypeStruct((batch, WINDOW, p.shape[-1]), p.dtype),
    )
    run = pl.kernel(
        functools.partial(_sc_body, start),
        mesh=mesh,
        out_type=out_type,
        scratch_types=[pltpu.SemaphoreType.DMA],
        compiler_params=pltpu.CompilerParams(use_tc_tiling_on_sc=False),
    )
    return run(ddx, mdx, p)


# speedup vs baseline: 9.7067x; 9.7067x over previous
import functools

import jax
import jax.numpy as jnp
import numpy as np
from jax import lax
from jax.experimental import pallas as pl
from jax.experimental.pallas import tpu as pltpu
from jax.experimental.pallas import tpu_sc as plsc

WINDOW = 2048
ROWS = 32
NCHUNK = WINDOW // ROWS


def _window_start(max_len: int) -> int:
    if max_len == WINDOW:
        return 0
    return int(np.random.RandomState(0).randint(0, max_len - WINDOW + 1))


def _sc_body(start, ddx, mdx, p, oddx, omdx, op_, buf, pbuf,
             semi0, semi1, semo0, semo1):
    cid = lax.axis_index("c")
    b = lax.axis_index("s")
    semi = [semi0, semi1]
    semo = [semo0, semo1]

    def pump(src, dst):
        def chunk_in(i, t):
            return pltpu.make_async_copy(
                src.at[b, pl.ds(start + i * ROWS, ROWS)], buf.at[t], semi[t])

        def chunk_out(i, t):
            return pltpu.make_async_copy(
                buf.at[t], dst.at[b, pl.ds(i * ROWS, ROWS)], semo[t])

        chunk_in(0, 0).start()
        chunk_in(1, 1).start()

        def step(g, carry):
            for t in range(2):
                i = 2 * g + t
                chunk_in(i, t).wait()
                chunk_out(i, t).start()
            for t in range(2):
                i = 2 * g + t
                chunk_out(i, t).wait()
                nxt = i + 2

                @pl.when(nxt < NCHUNK)
                def _():
                    chunk_in(nxt, t).start()

            return carry

        lax.fori_loop(0, NCHUNK // 2, step, 0)

    @pl.when(cid == 0)
    def _():
        pump(ddx, oddx)
        pin = pltpu.make_async_copy(p.at[b, pl.ds(start, WINDOW)], pbuf, semi0)
        pin.start()
        pin.wait()
        pout = pltpu.make_async_copy(pbuf, op_.at[b], semo0)
        pout.start()
        pout.wait()

    @pl.when(cid == 1)
    def _():
        pump(mdx, omdx)


@jax.jit
def kernel(ddx, mdx, p):
    batch, max_len, dm = ddx.shape
    start = _window_start(max_len)
    mesh = plsc.VectorSubcoreMesh(core_axis_name="c", subcore_axis_name="s")
    out_type = (
        jax.ShapeDtypeStruct((batch, WINDOW, dm), ddx.dtype),
        jax.ShapeDtypeStruct((batch, WINDOW, dm), mdx.dtype),
        jax.ShapeDtypeStruct((batch, WINDOW, p.shape[-1]), p.dtype),
    )
    run = pl.kernel(
        functools.partial(_sc_body, start),
        mesh=mesh,
        out_type=out_type,
        scratch_types=[
            pltpu.VMEM((2, ROWS, dm), ddx.dtype),
            pltpu.VMEM((WINDOW, p.shape[-1]), p.dtype),
            pltpu.SemaphoreType.DMA,
            pltpu.SemaphoreType.DMA,
            pltpu.SemaphoreType.DMA,
            pltpu.SemaphoreType.DMA,
        ],
        compiler_params=pltpu.CompilerParams(use_tc_tiling_on_sc=False),
    )
    return run(ddx, mdx, p)


# speedup vs baseline: 42.4576x; 4.3740x over previous
import functools

import jax
import jax.numpy as jnp
import numpy as np
from jax.experimental import pallas as pl
from jax.experimental.pallas import tpu as pltpu

WINDOW = 2048
CHUNK = 512
K = WINDOW // CHUNK


def _window_start(max_len: int) -> int:
    if max_len == WINDOW:
        return 0
    return int(np.random.RandomState(0).randint(0, max_len - WINDOW + 1))


def _tc_body(start, nb, ddx, mdx, p, oddx_ref, omdx_ref, op_ref,
             bufd, bufm, bufp, semd, semm, semp):
    off = start % 8
    base = start - off
    b = pl.program_id(0)
    k = pl.program_id(1)
    g = b * K + k

    def start_dmas(bb, kk, slot):
        row = base + kk * CHUNK
        pltpu.make_async_copy(
            ddx.at[bb, pl.ds(row, CHUNK + 8)], bufd.at[slot], semd.at[slot]
        ).start()
        pltpu.make_async_copy(
            mdx.at[bb, pl.ds(row, CHUNK + 8)], bufm.at[slot], semm.at[slot]
        ).start()
        pltpu.make_async_copy(
            p.at[bb, pl.ds(row, CHUNK + 8)], bufp.at[slot], semp.at[slot]
        ).start()

    @pl.when(g == 0)
    def _():
        start_dmas(0, 0, 0)

    @pl.when(g + 1 < nb * K)
    def _():
        nk = (k + 1) % K
        nbb = b + (k + 1) // K
        start_dmas(nbb, nk, (g + 1) % 2)

    slot = g % 2
    row = base + k * CHUNK
    pltpu.make_async_copy(
        ddx.at[b, pl.ds(row, CHUNK + 8)], bufd.at[slot], semd.at[slot]
    ).wait()
    pltpu.make_async_copy(
        mdx.at[b, pl.ds(row, CHUNK + 8)], bufm.at[slot], semm.at[slot]
    ).wait()
    pltpu.make_async_copy(
        p.at[b, pl.ds(row, CHUNK + 8)], bufp.at[slot], semp.at[slot]
    ).wait()

    oddx_ref[0] = bufd[slot, pl.ds(off, CHUNK), :]
    omdx_ref[0] = bufm[slot, pl.ds(off, CHUNK), :]
    op_ref[0] = bufp[slot, pl.ds(off, CHUNK), :]


@jax.jit
def kernel(ddx, mdx, p):
    batch, max_len, dm = ddx.shape
    dp = p.shape[-1]
    start = _window_start(max_len)
    grid = (batch, K)
    out_shape = (
        jax.ShapeDtypeStruct((batch, WINDOW, dm), ddx.dtype),
        jax.ShapeDtypeStruct((batch, WINDOW, dm), mdx.dtype),
        jax.ShapeDtypeStruct((batch, WINDOW, dp), p.dtype),
    )
    return pl.pallas_call(
        functools.partial(_tc_body, start, batch),
        grid=grid,
        in_specs=[
            pl.BlockSpec(memory_space=pl.ANY),
            pl.BlockSpec(memory_space=pl.ANY),
            pl.BlockSpec(memory_space=pl.ANY),
        ],
        out_specs=(
            pl.BlockSpec((1, CHUNK, dm), lambda b, k: (b, k, 0)),
            pl.BlockSpec((1, CHUNK, dm), lambda b, k: (b, k, 0)),
            pl.BlockSpec((1, CHUNK, dp), lambda b, k: (b, k, 0)),
        ),
        out_shape=out_shape,
        scratch_shapes=[
            pltpu.VMEM((2, CHUNK + 8, dm), ddx.dtype),
            pltpu.VMEM((2, CHUNK + 8, dm), mdx.dtype),
            pltpu.VMEM((2, CHUNK + 8, dp), p.dtype),
            pltpu.SemaphoreType.DMA((2,)),
            pltpu.SemaphoreType.DMA((2,)),
            pltpu.SemaphoreType.DMA((2,)),
        ],
        compiler_params=pltpu.CompilerParams(
            dimension_semantics=("arbitrary", "arbitrary"),
        ),
    )(ddx, mdx, p)


# speedup vs baseline: 43.1697x; 1.0168x over previous
import functools

import jax
import jax.numpy as jnp
import numpy as np
from jax.experimental import pallas as pl
from jax.experimental.pallas import tpu as pltpu

WINDOW = 2048
CHUNK = 1024
K = WINDOW // CHUNK


def _window_start(max_len: int) -> int:
    if max_len == WINDOW:
        return 0
    return int(np.random.RandomState(0).randint(0, max_len - WINDOW + 1))


def _tc_body(start, nb, ddx, mdx, p, oddx_ref, omdx_ref, op_ref,
             bufd, bufm, bufp, semd, semm, semp):
    off = start % 8
    base = start - off
    b = pl.program_id(0)
    k = pl.program_id(1)
    g = b * K + k

    def start_dmas(bb, kk, slot):
        row = base + kk * CHUNK
        pltpu.make_async_copy(
            ddx.at[bb, pl.ds(row, CHUNK + 8)], bufd.at[slot], semd.at[slot]
        ).start()
        pltpu.make_async_copy(
            mdx.at[bb, pl.ds(row, CHUNK + 8)], bufm.at[slot], semm.at[slot]
        ).start()
        pltpu.make_async_copy(
            p.at[bb, pl.ds(row, CHUNK + 8)], bufp.at[slot], semp.at[slot]
        ).start()

    @pl.when(g == 0)
    def _():
        start_dmas(0, 0, 0)

    @pl.when(g + 1 < nb * K)
    def _():
        nk = (k + 1) % K
        nbb = b + (k + 1) // K
        start_dmas(nbb, nk, (g + 1) % 2)

    slot = g % 2
    row = base + k * CHUNK
    pltpu.make_async_copy(
        ddx.at[b, pl.ds(row, CHUNK + 8)], bufd.at[slot], semd.at[slot]
    ).wait()
    pltpu.make_async_copy(
        mdx.at[b, pl.ds(row, CHUNK + 8)], bufm.at[slot], semm.at[slot]
    ).wait()
    pltpu.make_async_copy(
        p.at[b, pl.ds(row, CHUNK + 8)], bufp.at[slot], semp.at[slot]
    ).wait()

    oddx_ref[0] = bufd[slot, pl.ds(off, CHUNK), :]
    omdx_ref[0] = bufm[slot, pl.ds(off, CHUNK), :]
    op_ref[0] = bufp[slot, pl.ds(off, CHUNK), :]


@jax.jit
def kernel(ddx, mdx, p):
    batch, max_len, dm = ddx.shape
    dp = p.shape[-1]
    start = _window_start(max_len)
    grid = (batch, K)
    out_shape = (
        jax.ShapeDtypeStruct((batch, WINDOW, dm), ddx.dtype),
        jax.ShapeDtypeStruct((batch, WINDOW, dm), mdx.dtype),
        jax.ShapeDtypeStruct((batch, WINDOW, dp), p.dtype),
    )
    return pl.pallas_call(
        functools.partial(_tc_body, start, batch),
        grid=grid,
        in_specs=[
            pl.BlockSpec(memory_space=pl.ANY),
            pl.BlockSpec(memory_space=pl.ANY),
            pl.BlockSpec(memory_space=pl.ANY),
        ],
        out_specs=(
            pl.BlockSpec((1, CHUNK, dm), lambda b, k: (b, k, 0)),
            pl.BlockSpec((1, CHUNK, dm), lambda b, k: (b, k, 0)),
            pl.BlockSpec((1, CHUNK, dp), lambda b, k: (b, k, 0)),
        ),
        out_shape=out_shape,
        scratch_shapes=[
            pltpu.VMEM((2, CHUNK + 8, dm), ddx.dtype),
            pltpu.VMEM((2, CHUNK + 8, dm), mdx.dtype),
            pltpu.VMEM((2, CHUNK + 8, dp), p.dtype),
            pltpu.SemaphoreType.DMA((2,)),
            pltpu.SemaphoreType.DMA((2,)),
            pltpu.SemaphoreType.DMA((2,)),
        ],
        compiler_params=pltpu.CompilerParams(
            dimension_semantics=("arbitrary", "arbitrary"),
        ),
    )(ddx, mdx, p)


# speedup vs baseline: 43.2782x; 1.0025x over previous
import functools

import jax
import jax.numpy as jnp
import numpy as np
from jax.experimental import pallas as pl
from jax.experimental.pallas import tpu as pltpu

WINDOW = 2048
CHUNK = 1024
K = WINDOW // CHUNK


def _window_start(max_len: int) -> int:
    if max_len == WINDOW:
        return 0
    return int(np.random.RandomState(0).randint(0, max_len - WINDOW + 1))


def _tc_body(start, nb, ddx, mdx, p, oddx_ref, omdx_ref, op_ref,
             bufd, bufm, bufp, semd, semm, semp):
    off = start % 8
    base = start - off
    b = pl.program_id(0)
    k = pl.program_id(1)
    g = b * K + k

    def start_dmas(bb, kk, slot):
        row = base + kk * CHUNK
        pltpu.make_async_copy(
            ddx.at[bb, pl.ds(row, CHUNK + 8)], bufd.at[slot], semd.at[slot]
        ).start()
        pltpu.make_async_copy(
            mdx.at[bb, pl.ds(row, CHUNK + 8)], bufm.at[slot], semm.at[slot]
        ).start()
        pltpu.make_async_copy(
            p.at[bb, pl.ds(row, CHUNK + 8)], bufp.at[slot], semp.at[slot]
        ).start()

    @pl.when(g == 0)
    def _():
        start_dmas(0, 0, 0)
        start_dmas(0, 1, 1)

    @pl.when(g + 2 < nb * K)
    def _():
        nk = (k + 2) % K
        nbb = b + (k + 2) // K
        start_dmas(nbb, nk, (g + 2) % 3)

    slot = g % 3
    row = base + k * CHUNK
    pltpu.make_async_copy(
        ddx.at[b, pl.ds(row, CHUNK + 8)], bufd.at[slot], semd.at[slot]
    ).wait()
    pltpu.make_async_copy(
        mdx.at[b, pl.ds(row, CHUNK + 8)], bufm.at[slot], semm.at[slot]
    ).wait()
    pltpu.make_async_copy(
        p.at[b, pl.ds(row, CHUNK + 8)], bufp.at[slot], semp.at[slot]
    ).wait()

    oddx_ref[0] = bufd[slot, pl.ds(off, CHUNK), :]
    omdx_ref[0] = bufm[slot, pl.ds(off, CHUNK), :]
    op_ref[0] = bufp[slot, pl.ds(off, CHUNK), :]


@jax.jit
def kernel(ddx, mdx, p):
    batch, max_len, dm = ddx.shape
    dp = p.shape[-1]
    start = _window_start(max_len)
    grid = (batch, K)
    out_shape = (
        jax.ShapeDtypeStruct((batch, WINDOW, dm), ddx.dtype),
        jax.ShapeDtypeStruct((batch, WINDOW, dm), mdx.dtype),
        jax.ShapeDtypeStruct((batch, WINDOW, dp), p.dtype),
    )
    return pl.pallas_call(
        functools.partial(_tc_body, start, batch),
        grid=grid,
        in_specs=[
            pl.BlockSpec(memory_space=pl.ANY),
            pl.BlockSpec(memory_space=pl.ANY),
            pl.BlockSpec(memory_space=pl.ANY),
        ],
        out_specs=(
            pl.BlockSpec((1, CHUNK, dm), lambda b, k: (b, k, 0)),
            pl.BlockSpec((1, CHUNK, dm), lambda b, k: (b, k, 0)),
            pl.BlockSpec((1, CHUNK, dp), lambda b, k: (b, k, 0)),
        ),
        out_shape=out_shape,
        scratch_shapes=[
            pltpu.VMEM((3, CHUNK + 8, dm), ddx.dtype),
            pltpu.VMEM((3, CHUNK + 8, dm), mdx.dtype),
            pltpu.VMEM((3, CHUNK + 8, dp), p.dtype),
            pltpu.SemaphoreType.DMA((3,)),
            pltpu.SemaphoreType.DMA((3,)),
            pltpu.SemaphoreType.DMA((3,)),
        ],
        compiler_params=pltpu.CompilerParams(
            dimension_semantics=("arbitrary", "arbitrary"),
        ),
    )(ddx, mdx, p)
